# baseline (speedup 1.0000x reference)
import functools

import jax
import jax.numpy as jnp
from jax import lax
from jax.experimental import pallas as pl
from jax.experimental.pallas import tpu as pltpu
from jax.experimental.pallas import tpu_sc as plsc

N = 10000
E = 320000
D = 128

NPAD = 10240
HALF = NPAD // 2
ACC_ROWS = 5376
CB = 128
NCDEG = 80
NCEDGE = 160
EPAD = 32 * NCDEG * CB
EROWS = EPAD // CB
DEG_RPT = NPAD // 16
ACC_RPT = ACC_ROWS // 16
NBUF = 2
NGROUP = NCEDGE // NBUF

_mesh = plsc.VectorSubcoreMesh(core_axis_name="c", subcore_axis_name="s")


@functools.partial(
    pl.kernel,
    mesh=_mesh,
    out_type=jax.ShapeDtypeStruct((2, NPAD, 16), jnp.float32),
    scratch_types=[
        pltpu.VMEM((NCDEG, CB), jnp.int32),
        pltpu.VMEM((CB, 16), jnp.float32),
        pltpu.VMEM((64, 16), jnp.float32),
        pltpu.VMEM_SHARED((NPAD, 16), jnp.float32),
        pltpu.SemaphoreType.DMA,
    ],
)
def _deg_kernel(col_hbm, deg_out, col_v, ones_v, zero_v, deg_sh, hsem):
    c = lax.axis_index("c")
    s = lax.axis_index("s")
    wid = s * 2 + c
    for i in range(64):
        zero_v[i, pl.ds(0, 16)] = jnp.zeros((16,), jnp.float32)
    for i in range(CB):
        ones_v[i, pl.ds(0, 16)] = jnp.ones((16,), jnp.float32)

    def zero_body(j, carry):
        pltpu.sync_copy(zero_v, deg_sh.at[pl.ds(s * DEG_RPT + j * 64, 64)])
        return carry

    lax.fori_loop(0, DEG_RPT // 64, zero_body, 0)
    pltpu.sync_copy(col_hbm.at[pl.ds(wid * NCDEG, NCDEG)], col_v)
    plsc.subcore_barrier()

    def hist_body(j, carry):
        pltpu.async_copy(ones_v, deg_sh.at[col_v.at[j]], hsem, add=True)
        return carry

    lax.fori_loop(0, NCDEG, hist_body, 0)

    def hist_drain(j, carry):
        pltpu.make_async_copy(ones_v, deg_sh.at[col_v.at[j]], hsem).wait()
        return carry

    lax.fori_loop(0, NCDEG, hist_drain, 0)
    plsc.subcore_barrier()
    pltpu.sync_copy(deg_sh.at[pl.ds(s * DEG_RPT, DEG_RPT)],
                    deg_out.at[c, pl.ds(s * DEG_RPT, DEG_RPT)])


@functools.partial(
    pl.kernel,
    mesh=_mesh,
    out_type=jax.ShapeDtypeStruct((2, ACC_ROWS, D), jnp.float32),
    scratch_types=[
        pltpu.VMEM((NCEDGE, CB), jnp.int32),
        pltpu.VMEM((NCEDGE, CB), jnp.int32),
        pltpu.VMEM((CB, D), jnp.float32),
        pltpu.VMEM((CB, D), jnp.float32),
        pltpu.VMEM((16, D), jnp.float32),
        pltpu.VMEM_SHARED((ACC_ROWS, D), jnp.float32),
        pltpu.SemaphoreType.DMA,
        pltpu.SemaphoreType.DMA,
        pltpu.SemaphoreType.DMA,
        pltpu.SemaphoreType.DMA,
    ],
)
def _edge_kernel(row_hbm, col_hbm, t_hbm, out_hbm,
                 row_v, col_v, b0, b1, zero_v, acc_sh,
                 g0, g1, s0, s1):
    c = lax.axis_index("c")
    s = lax.axis_index("s")
    bufs = [b0, b1]
    gs = [g0, g1]
    ss = [s0, s1]

    for i in range(16):
        for d in range(D // 16):
            zero_v[i, pl.ds(d * 16, 16)] = jnp.zeros((16,), jnp.float32)

    def zero_body(j, carry):
        pltpu.sync_copy(zero_v, acc_sh.at[pl.ds(s * ACC_RPT + j * 16, 16)])
        return carry

    lax.fori_loop(0, ACC_RPT // 16, zero_body, 0)
    pltpu.sync_copy(row_hbm.at[pl.ds(s * NCEDGE, NCEDGE)], row_v)
    pltpu.sync_copy(col_hbm.at[c, pl.ds(s * NCEDGE, NCEDGE)], col_v)

    def gather(ch, k):
        pltpu.async_copy(t_hbm.at[row_v.at[ch]], bufs[k], gs[k])

    def gather_wait(ch, k):
        pltpu.make_async_copy(t_hbm.at[row_v.at[ch]], bufs[k], gs[k]).wait()

    for k in range(NBUF):
        gather(k, k)
    plsc.subcore_barrier()

    def body(g, carry):
        base = NBUF * g
        for k in range(NBUF):
            gather_wait(base + k, k)
            pltpu.async_copy(bufs[k], acc_sh.at[col_v.at[base + k]], ss[k],
                             add=True)
        for k in range(NBUF):
            pltpu.make_async_copy(bufs[k], acc_sh.at[col_v.at[base + k]],
                                  ss[k]).wait()
            gather(jnp.minimum(base + k + NBUF, NCEDGE - 1), k)
        return carry

    lax.fori_loop(0, NGROUP, body, 0)
    for k in range(NBUF):
        gather_wait(NCEDGE - 1, k)
    plsc.subcore_barrier()

    def out_body(j, carry):
        pltpu.sync_copy(acc_sh.at[pl.ds(s * ACC_RPT + j * 16, 16)],
                        out_hbm.at[c, pl.ds(s * ACC_RPT + j * 16, 16)])
        return carry

    lax.fori_loop(0, ACC_RPT // 16, out_body, 0)


def _tc_first_body(x_ref, w_ref, degp_ref, t_ref, dis_ref):
    deg = degp_ref[0, :, 0] + degp_ref[1, :, 0] + 1.0
    dis = lax.rsqrt(deg)
    t = jnp.dot(x_ref[...], w_ref[...], preferred_element_type=jnp.float32,
                precision=lax.Precision.HIGHEST)
    t_ref[...] = t * dis[:, None]
    dis_ref[...] = dis


def _tc_mid_body(p_ref, tp_ref, dis_ref, b_ref, w_ref, out_ref):
    dis = dis_ref[...]
    h = dis[:, None] * (p_ref[0] + tp_ref[...]) + b_ref[...]
    t = jnp.dot(h, w_ref[...], preferred_element_type=jnp.float32,
                precision=lax.Precision.HIGHEST)
    out_ref[...] = t * dis[:, None]


def _tc_final_body(p_ref, tp_ref, dis_ref, b_ref, out_ref):
    dis = dis_ref[...]
    out_ref[...] = dis[:, None] * (p_ref[0] + tp_ref[...]) + b_ref[...]


_BN = 1024
_GRID = NPAD // _BN
_NBH = HALF // _BN


def _p_map(i):
    return (i // _NBH, i % _NBH, 0)


_tc_first = pl.pallas_call(
    _tc_first_body,
    grid=(_GRID,),
    in_specs=[
        pl.BlockSpec((_BN, D), lambda i: (i, 0)),
        pl.BlockSpec((D, D), lambda i: (0, 0)),
        pl.BlockSpec((2, _BN, 16), lambda i: (0, i, 0)),
    ],
    out_specs=[
        pl.BlockSpec((_BN, D), lambda i: (i, 0)),
        pl.BlockSpec((_BN,), lambda i: (i,)),
    ],
    out_shape=[
        jax.ShapeDtypeStruct((NPAD, D), jnp.float32),
        jax.ShapeDtypeStruct((NPAD,), jnp.float32),
    ],
)

_tc_mid = pl.pallas_call(
    _tc_mid_body,
    grid=(_GRID,),
    in_specs=[
        pl.BlockSpec((1, _BN, D), _p_map),
        pl.BlockSpec((_BN, D), lambda i: (i, 0)),
        pl.BlockSpec((_BN,), lambda i: (i,)),
        pl.BlockSpec((1, D), lambda i: (0, 0)),
        pl.BlockSpec((D, D), lambda i: (0, 0)),
    ],
    out_specs=pl.BlockSpec((_BN, D), lambda i: (i, 0)),
    out_shape=jax.ShapeDtypeStruct((NPAD, D), jnp.float32),
)

_tc_final = pl.pallas_call(
    _tc_final_body,
    grid=(_GRID,),
    in_specs=[
        pl.BlockSpec((1, _BN, D), _p_map),
        pl.BlockSpec((_BN, D), lambda i: (i, 0)),
        pl.BlockSpec((_BN,), lambda i: (i,)),
        pl.BlockSpec((1, D), lambda i: (0, 0)),
    ],
    out_specs=pl.BlockSpec((_BN, D), lambda i: (i, 0)),
    out_shape=jax.ShapeDtypeStruct((NPAD, D), jnp.float32),
)


def kernel(x, edge_index, W1, b1, W2, b2, W3, b3):
    row = edge_index[0]
    col = edge_index[1]
    pad = jnp.full((EPAD - E,), N, jnp.int32)
    rowp = jnp.concatenate([row, pad])
    colp = jnp.concatenate([col, pad])
    col_c0 = jnp.where(colp < HALF, colp, HALF)
    col_c1 = jnp.where((colp >= HALF) & (colp < N), colp - HALF, HALF)
    col2 = jnp.stack([col_c0, col_c1]).reshape(2, EROWS, CB)
    row2d = rowp.reshape(EROWS, CB)
    coldeg = colp.reshape(EROWS, CB)
    xpad = jnp.concatenate([x, jnp.zeros((NPAD - N, D), jnp.float32)])

    degp = _deg_kernel(coldeg)
    t1, dis = _tc_first(xpad, W1, degp)
    s1 = _edge_kernel(row2d, col2, t1)
    t2 = _tc_mid(s1, t1, dis, b1.reshape(1, D), W2)
    s2 = _edge_kernel(row2d, col2, t2)
    t3 = _tc_mid(s2, t2, dis, b2.reshape(1, D), W3)
    s3 = _edge_kernel(row2d, col2, t3)
    out = _tc_final(s3, t3, dis, b3.reshape(1, D))
    return out[:N]


# speedup vs baseline: 1.5403x; 1.5403x over previous
import functools

import jax
import jax.numpy as jnp
from jax import lax
from jax.experimental import pallas as pl
from jax.experimental.pallas import tpu as pltpu
from jax.experimental.pallas import tpu_sc as plsc

N = 10000
E = 320000
D = 128

NPAD = 10240
CB = 128
BIG = 256
NCDEG = 80
EPT = NCDEG * CB
EPAD = 32 * EPT
EROWS = EPAD // CB
NBIG = EPT // BIG
RPT = NPAD // 16

_mesh = plsc.VectorSubcoreMesh(core_axis_name="c", subcore_axis_name="s")


@functools.partial(
    pl.kernel,
    mesh=_mesh,
    out_type=jax.ShapeDtypeStruct((2, NPAD, 16), jnp.float32),
    scratch_types=[
        pltpu.VMEM((NCDEG, CB), jnp.int32),
        pltpu.VMEM((CB, 16), jnp.float32),
        pltpu.VMEM((64, 16), jnp.float32),
        pltpu.VMEM_SHARED((NPAD, 16), jnp.float32),
        pltpu.SemaphoreType.DMA,
    ],
)
def _deg_kernel(col_hbm, deg_out, col_v, ones_v, zero_v, deg_sh, hsem):
    c = lax.axis_index("c")
    s = lax.axis_index("s")
    wid = s * 2 + c
    for i in range(64):
        zero_v[i, pl.ds(0, 16)] = jnp.zeros((16,), jnp.float32)
    for i in range(CB):
        ones_v[i, pl.ds(0, 16)] = jnp.ones((16,), jnp.float32)

    def zero_body(j, carry):
        pltpu.sync_copy(zero_v, deg_sh.at[pl.ds(s * RPT + j * 64, 64)])
        return carry

    lax.fori_loop(0, RPT // 64, zero_body, 0)
    pltpu.sync_copy(col_hbm.at[pl.ds(wid * NCDEG, NCDEG)], col_v)
    plsc.subcore_barrier()

    def hist_body(j, carry):
        pltpu.async_copy(ones_v, deg_sh.at[col_v.at[j]], hsem, add=True)
        return carry

    lax.fori_loop(0, NCDEG, hist_body, 0)

    def hist_drain(j, carry):
        pltpu.make_async_copy(ones_v, deg_sh.at[col_v.at[j]], hsem).wait()
        return carry

    lax.fori_loop(0, NCDEG, hist_drain, 0)
    plsc.subcore_barrier()
    pltpu.sync_copy(deg_sh.at[pl.ds(s * RPT, RPT)],
                    deg_out.at[c, pl.ds(s * RPT, RPT)])


@functools.partial(
    pl.kernel,
    mesh=_mesh,
    out_type=jax.ShapeDtypeStruct((2, NPAD, D), jnp.float32),
    scratch_types=[
        pltpu.VMEM((NCDEG, CB), jnp.int32),
        pltpu.VMEM((NCDEG, CB), jnp.int32),
        pltpu.VMEM((CB, D), jnp.float32),
        pltpu.VMEM((16, D), jnp.float32),
        pltpu.VMEM_SHARED((NPAD, D), jnp.float32),
        pltpu.SemaphoreType.DMA,
        pltpu.SemaphoreType.DMA,
    ],
)
def _edge_kernel(row_hbm, col_hbm, t_hbm, out_hbm,
                 row_v, col_v, bb, zero_v, acc_sh, gsem, ssem):
    c = lax.axis_index("c")
    s = lax.axis_index("s")
    wid = s * 2 + c

    for i in range(16):
        for d in range(D // 16):
            zero_v[i, pl.ds(d * 16, 16)] = jnp.zeros((16,), jnp.float32)

    def zero_body(j, carry):
        pltpu.sync_copy(zero_v, acc_sh.at[pl.ds(s * RPT + j * 16, 16)])
        return carry

    lax.fori_loop(0, RPT // 16, zero_body, 0)
    pltpu.sync_copy(row_hbm.at[pl.ds(wid * NCDEG, NCDEG)], row_v)
    pltpu.sync_copy(col_hbm.at[pl.ds(wid * NCDEG, NCDEG)], col_v)
    plsc.subcore_barrier()

    def body(g, carry):
        pltpu.async_copy(t_hbm.at[row_v.at[g]], bb, gsem).wait()
        pltpu.sync_copy(bb, acc_sh.at[col_v.at[g]], add=True)
        return carry

    lax.fori_loop(0, NCDEG, body, 0)
    plsc.subcore_barrier()

    def out_body(j, carry):
        pltpu.sync_copy(acc_sh.at[pl.ds(s * RPT + j * 16, 16)],
                        out_hbm.at[c, pl.ds(s * RPT + j * 16, 16)])
        return carry

    lax.fori_loop(0, RPT // 16, out_body, 0)


def _tc_first_body(x_ref, w_ref, degp_ref, t_ref, dis_ref):
    deg = degp_ref[0, :, 0] + degp_ref[1, :, 0] + 1.0
    dis = lax.rsqrt(deg)
    t = jnp.dot(x_ref[...], w_ref[...], preferred_element_type=jnp.float32,
                precision=lax.Precision.HIGHEST)
    t_ref[...] = t * dis[:, None]
    dis_ref[...] = dis


def _tc_mid_body(p_ref, tp_ref, dis_ref, b_ref, w_ref, out_ref):
    dis = dis_ref[...]
    h = dis[:, None] * (p_ref[0] + p_ref[1] + tp_ref[...]) + b_ref[...]
    t = jnp.dot(h, w_ref[...], preferred_element_type=jnp.float32,
                precision=lax.Precision.HIGHEST)
    out_ref[...] = t * dis[:, None]


def _tc_final_body(p_ref, tp_ref, dis_ref, b_ref, out_ref):
    dis = dis_ref[...]
    out_ref[...] = dis[:, None] * (p_ref[0] + p_ref[1] + tp_ref[...]) + b_ref[...]


_BN = 1024
_GRID = NPAD // _BN

_tc_first = pl.pallas_call(
    _tc_first_body,
    grid=(_GRID,),
    in_specs=[
        pl.BlockSpec((_BN, D), lambda i: (i, 0)),
        pl.BlockSpec((D, D), lambda i: (0, 0)),
        pl.BlockSpec((2, _BN, 16), lambda i: (0, i, 0)),
    ],
    out_specs=[
        pl.BlockSpec((_BN, D), lambda i: (i, 0)),
        pl.BlockSpec((_BN,), lambda i: (i,)),
    ],
    out_shape=[
        jax.ShapeDtypeStruct((NPAD, D), jnp.float32),
        jax.ShapeDtypeStruct((NPAD,), jnp.float32),
    ],
)

_tc_mid = pl.pallas_call(
    _tc_mid_body,
    grid=(_GRID,),
    in_specs=[
        pl.BlockSpec((2, _BN, D), lambda i: (0, i, 0)),
        pl.BlockSpec((_BN, D), lambda i: (i, 0)),
        pl.BlockSpec((_BN,), lambda i: (i,)),
        pl.BlockSpec((1, D), lambda i: (0, 0)),
        pl.BlockSpec((D, D), lambda i: (0, 0)),
    ],
    out_specs=pl.BlockSpec((_BN, D), lambda i: (i, 0)),
    out_shape=jax.ShapeDtypeStruct((NPAD, D), jnp.float32),
)

_tc_final = pl.pallas_call(
    _tc_final_body,
    grid=(_GRID,),
    in_specs=[
        pl.BlockSpec((2, _BN, D), lambda i: (0, i, 0)),
        pl.BlockSpec((_BN, D), lambda i: (i, 0)),
        pl.BlockSpec((_BN,), lambda i: (i,)),
        pl.BlockSpec((1, D), lambda i: (0, 0)),
    ],
    out_specs=pl.BlockSpec((_BN, D), lambda i: (i, 0)),
    out_shape=jax.ShapeDtypeStruct((NPAD, D), jnp.float32),
)


def kernel(x, edge_index, W1, b1, W2, b2, W3, b3):
    row = edge_index[0]
    col = edge_index[1]
    pad = jnp.full((EPAD - E,), N, jnp.int32)
    rowp = jnp.concatenate([row, pad])
    colp = jnp.concatenate([col, pad])
    col2d = colp.reshape(EROWS, CB)
    row2d = rowp.reshape(EROWS, CB)
    xpad = jnp.concatenate([x, jnp.zeros((NPAD - N, D), jnp.float32)])

    degp = _deg_kernel(col2d)
    t1, dis = _tc_first(xpad, W1, degp)
    s1 = _edge_kernel(row2d, col2d, t1)
    t2 = _tc_mid(s1, t1, dis, b1.reshape(1, D), W2)
    s2 = _edge_kernel(row2d, col2d, t2)
    t3 = _tc_mid(s2, t2, dis, b2.reshape(1, D), W3)
    s3 = _edge_kernel(row2d, col2d, t3)
    out = _tc_final(s3, t3, dis, b3.reshape(1, D))
    return out[:N]
